# baseline (speedup 1.0000x reference)
import jax
import jax.numpy as jnp
from jax.experimental import pallas as pl
from jax.experimental.pallas import tpu as pltpu

_N = 10000
_K = 32
_R = 0.1 * 0.999
_R2 = _R * _R

_BR = 128
_W = 10240
_GRID = _W // _BR
_S = _W // 128
_BS = 4
_T = 10
_RG = 8


def _insert(v, ci, vals, idxs):
    new_vals, new_idxs = [], []
    c_prev = None
    for t in range(_T):
        c_t = v < vals[t]
        if t == 0:
            nv = jnp.where(c_t, v, vals[t])
            ni = jnp.where(c_t, ci, idxs[t])
        else:
            nv = jnp.where(c_t, jnp.where(c_prev, vals[t - 1], v), vals[t])
            ni = jnp.where(c_t, jnp.where(c_prev, idxs[t - 1], ci), idxs[t])
        new_vals.append(nv)
        new_idxs.append(ni)
        c_prev = c_t
    return tuple(new_vals), tuple(new_idxs)


def _pop(vals, idxs, r2):
    mval = jnp.min(vals[0], axis=1, keepdims=True)
    is_min = vals[0] == mval
    li = jnp.min(jnp.where(is_min, idxs[0], _W), axis=1, keepdims=True)
    pop = is_min & (idxs[0] == li)
    src_t = jnp.where(mval <= r2, li, -1)
    new_vals = tuple(jnp.where(pop, vals[u + 1], vals[u]) for u in range(_T - 1)) \
        + (jnp.where(pop, jnp.inf, vals[_T - 1]),)
    new_idxs = tuple(jnp.where(pop, idxs[u + 1], idxs[u]) for u in range(_T - 1)) \
        + (jnp.where(pop, _W, idxs[_T - 1]),)
    return src_t, new_vals, new_idxs


def _radius_topk_kernel(pos_r_ref, pos_t_ref, src_ref, dst_ref, work_ref):
    i = pl.program_id(0)
    pos_r = pos_r_ref[...]
    pos_t = pos_t_ref[...]
    sq_r = jnp.sum(pos_r * pos_r, axis=1, keepdims=True)
    sq_c = jnp.sum(pos_t * pos_t, axis=0, keepdims=True)
    m = jnp.dot(pos_r, pos_t, preferred_element_type=jnp.float32)
    d2 = sq_r + sq_c - 2.0 * m
    d2 = jnp.maximum(d2, 0.0)

    col = jax.lax.broadcasted_iota(jnp.int32, (_BR, _W), 1)
    row = jax.lax.broadcasted_iota(jnp.int32, (_BR, _W), 0) + i * _BR
    r2 = jnp.float32(_R2)
    valid = (d2 <= r2) & (col != row) & (col < _N) & (row < _N)
    work_ref[...] = jnp.where(valid, d2, jnp.inf)

    lane = jax.lax.broadcasted_iota(jnp.int32, (_BR, 128), 1)
    kcol = jax.lax.broadcasted_iota(jnp.int32, (_BR, _K), 1)

    def ins_body(g, carry):
        va, ia = carry
        for j in range(_BS):
            s = g * _BS + j
            x = work_ref[:, pl.ds(s * 128, 128)]
            ci = s * 128 + lane
            va, ia = _insert(x, ci, va, ia)
        return va, ia

    vals0 = tuple(jnp.full((_BR, 128), jnp.inf, jnp.float32) for _ in range(_T))
    idxs0 = tuple(jnp.full((_BR, 128), _W, jnp.int32) for _ in range(_T))
    va, ia = jax.lax.fori_loop(0, _S // _BS, ins_body, (vals0, idxs0))

    def ext_body(t, carry):
        va, ia, acc = carry
        s_t, va, ia = _pop(va, ia, r2)
        acc = jnp.where(kcol == t, s_t, acc)
        return va, ia, acc

    acc0 = jnp.full((_BR, _K), -1, jnp.int32)
    _, _, acc = jax.lax.fori_loop(0, _K, ext_body, (va, ia, acc0))

    rowk = jax.lax.broadcasted_iota(jnp.int32, (_BR, _K), 0) + i * _BR
    src_ref[...] = acc
    dst_ref[...] = jnp.where(acc >= 0, rowk, -1)


def kernel(feature, pos):
    pos_pad = jnp.pad(pos, ((0, _W - _N), (0, 0)), constant_values=100.0)
    pos_t = pos_pad.T
    src, dst = pl.pallas_call(
        _radius_topk_kernel,
        grid=(_GRID,),
        in_specs=[
            pl.BlockSpec((_BR, 3), lambda i: (i, 0)),
            pl.BlockSpec((3, _W), lambda i: (0, 0)),
        ],
        out_specs=[
            pl.BlockSpec((_BR, _K), lambda i: (i, 0)),
            pl.BlockSpec((_BR, _K), lambda i: (i, 0)),
        ],
        out_shape=[
            jax.ShapeDtypeStruct((_W, _K), jnp.int32),
            jax.ShapeDtypeStruct((_W, _K), jnp.int32),
        ],
        scratch_shapes=[pltpu.VMEM((_BR, _W), jnp.float32)],
    )(pos_pad, pos_t)
    edge_src = src[:_N].reshape(-1)
    edge_dst = dst[:_N].reshape(-1)
    return feature, pos, edge_src, edge_dst


# speedup vs baseline: 1.1529x; 1.1529x over previous
import jax
import jax.numpy as jnp
from jax.experimental import pallas as pl
from jax.experimental.pallas import tpu as pltpu

_N = 10000
_K = 32
_R = 0.1 * 0.999
_R2 = _R * _R

_BR = 128
_W = 10240
_GRID = _W // _BR
_S = _W // 128
_BS = 8
_T = 10
_RG = 8


def _insert(v, ci, vals, idxs):
    new_vals, new_idxs = [], []
    c_prev = None
    for t in range(_T):
        c_t = v < vals[t]
        if t == 0:
            nv = jnp.where(c_t, v, vals[t])
            ni = jnp.where(c_t, ci, idxs[t])
        else:
            nv = jnp.where(c_t, jnp.where(c_prev, vals[t - 1], v), vals[t])
            ni = jnp.where(c_t, jnp.where(c_prev, idxs[t - 1], ci), idxs[t])
        new_vals.append(nv)
        new_idxs.append(ni)
        c_prev = c_t
    return tuple(new_vals), tuple(new_idxs)


def _pop(vals, idxs, r2):
    mval = jnp.min(vals[0], axis=1, keepdims=True)
    is_min = vals[0] == mval
    li = jnp.min(jnp.where(is_min, idxs[0], _W), axis=1, keepdims=True)
    pop = is_min & (idxs[0] == li)
    src_t = jnp.where(mval <= r2, li, -1)
    new_vals = tuple(jnp.where(pop, vals[u + 1], vals[u]) for u in range(_T - 1)) \
        + (jnp.where(pop, jnp.inf, vals[_T - 1]),)
    new_idxs = tuple(jnp.where(pop, idxs[u + 1], idxs[u]) for u in range(_T - 1)) \
        + (jnp.where(pop, _W, idxs[_T - 1]),)
    return src_t, new_vals, new_idxs


def _radius_topk_kernel(pos_r_ref, pos_t_ref, src_ref, dst_ref, work_ref):
    i = pl.program_id(0)
    pos_r = pos_r_ref[...]
    pos_t = pos_t_ref[...]
    sq_r = jnp.sum(pos_r * pos_r, axis=1, keepdims=True)
    sq_c = jnp.sum(pos_t * pos_t, axis=0, keepdims=True)
    m = jnp.dot(pos_r, pos_t, preferred_element_type=jnp.float32)
    d2 = sq_r + sq_c - 2.0 * m
    d2 = jnp.maximum(d2, 0.0)

    col = jax.lax.broadcasted_iota(jnp.int32, (_BR, _W), 1)
    row = jax.lax.broadcasted_iota(jnp.int32, (_BR, _W), 0) + i * _BR
    r2 = jnp.float32(_R2)
    valid = (d2 <= r2) & (col != row) & (col < _N) & (row < _N)
    work_ref[...] = jnp.where(valid, d2, jnp.inf)

    lane = jax.lax.broadcasted_iota(jnp.int32, (_BR, 128), 1)
    kcol = jax.lax.broadcasted_iota(jnp.int32, (_BR, _K), 1)

    def ins_body(g, carry):
        va, ia = carry
        for j in range(_BS):
            s = g * _BS + j
            x = work_ref[:, pl.ds(s * 128, 128)]
            ci = s * 128 + lane
            va, ia = _insert(x, ci, va, ia)
        return va, ia

    vals0 = tuple(jnp.full((_BR, 128), jnp.inf, jnp.float32) for _ in range(_T))
    idxs0 = tuple(jnp.full((_BR, 128), _W, jnp.int32) for _ in range(_T))
    va, ia = jax.lax.fori_loop(0, _S // _BS, ins_body, (vals0, idxs0))

    def ext_body(g, carry):
        va, ia, acc = carry
        for j in range(2):
            t = g * 2 + j
            s_t, va, ia = _pop(va, ia, r2)
            acc = jnp.where(kcol == t, s_t, acc)
        return va, ia, acc

    acc0 = jnp.full((_BR, _K), -1, jnp.int32)
    _, _, acc = jax.lax.fori_loop(0, _K // 2, ext_body, (va, ia, acc0))

    rowk = jax.lax.broadcasted_iota(jnp.int32, (_BR, _K), 0) + i * _BR
    src_ref[...] = acc
    dst_ref[...] = jnp.where(acc >= 0, rowk, -1)


def kernel(feature, pos):
    pos_pad = jnp.pad(pos, ((0, _W - _N), (0, 0)), constant_values=100.0)
    pos_t = pos_pad.T
    src, dst = pl.pallas_call(
        _radius_topk_kernel,
        grid=(_GRID,),
        in_specs=[
            pl.BlockSpec((_BR, 3), lambda i: (i, 0)),
            pl.BlockSpec((3, _W), lambda i: (0, 0)),
        ],
        out_specs=[
            pl.BlockSpec((_BR, _K), lambda i: (i, 0)),
            pl.BlockSpec((_BR, _K), lambda i: (i, 0)),
        ],
        out_shape=[
            jax.ShapeDtypeStruct((_W, _K), jnp.int32),
            jax.ShapeDtypeStruct((_W, _K), jnp.int32),
        ],
        scratch_shapes=[pltpu.VMEM((_BR, _W), jnp.float32)],
    )(pos_pad, pos_t)
    edge_src = src[:_N].reshape(-1)
    edge_dst = dst[:_N].reshape(-1)
    return feature, pos, edge_src, edge_dst


# speedup vs baseline: 1.2828x; 1.1127x over previous
import jax
import jax.numpy as jnp
from jax.experimental import pallas as pl
from jax.experimental.pallas import tpu as pltpu

_N = 10000
_K = 32
_R = 0.1 * 0.999
_R2 = _R * _R

_BR = 128
_W = 10240
_GRID = _W // _BR
_S = _W // 128
_BS = 16
_T = 10
_RG = 8


def _insert(v, ci, vals, idxs):
    new_vals, new_idxs = [], []
    c_prev = None
    for t in range(_T):
        c_t = v < vals[t]
        if t == 0:
            nv = jnp.where(c_t, v, vals[t])
            ni = jnp.where(c_t, ci, idxs[t])
        else:
            nv = jnp.where(c_t, jnp.where(c_prev, vals[t - 1], v), vals[t])
            ni = jnp.where(c_t, jnp.where(c_prev, idxs[t - 1], ci), idxs[t])
        new_vals.append(nv)
        new_idxs.append(ni)
        c_prev = c_t
    return tuple(new_vals), tuple(new_idxs)


def _pop(vals, idxs, r2):
    mval = jnp.min(vals[0], axis=1, keepdims=True)
    is_min = vals[0] == mval
    li = jnp.min(jnp.where(is_min, idxs[0], _W), axis=1, keepdims=True)
    pop = is_min & (idxs[0] == li)
    src_t = jnp.where(mval <= r2, li, -1)
    new_vals = tuple(jnp.where(pop, vals[u + 1], vals[u]) for u in range(_T - 1)) \
        + (jnp.where(pop, jnp.inf, vals[_T - 1]),)
    new_idxs = tuple(jnp.where(pop, idxs[u + 1], idxs[u]) for u in range(_T - 1)) \
        + (jnp.where(pop, _W, idxs[_T - 1]),)
    return src_t, new_vals, new_idxs


def _radius_topk_kernel(pos_r_ref, pos_t_ref, src_ref, dst_ref, work_ref):
    i = pl.program_id(0)
    pos_r = pos_r_ref[...]
    pos_t = pos_t_ref[...]
    sq_r = jnp.sum(pos_r * pos_r, axis=1, keepdims=True)
    sq_c = jnp.sum(pos_t * pos_t, axis=0, keepdims=True)
    m = jnp.dot(pos_r, pos_t, preferred_element_type=jnp.float32)
    d2 = sq_r + sq_c - 2.0 * m
    d2 = jnp.maximum(d2, 0.0)

    col = jax.lax.broadcasted_iota(jnp.int32, (_BR, _W), 1)
    row = jax.lax.broadcasted_iota(jnp.int32, (_BR, _W), 0) + i * _BR
    r2 = jnp.float32(_R2)
    valid = (d2 <= r2) & (col != row)
    work_ref[...] = jnp.where(valid, d2, jnp.inf)

    lane = jax.lax.broadcasted_iota(jnp.int32, (_BR, 128), 1)
    kcol = jax.lax.broadcasted_iota(jnp.int32, (_BR, _K), 1)

    def ins_body(g, carry):
        va, ia = carry
        for j in range(_BS):
            s = g * _BS + j
            x = work_ref[:, pl.ds(s * 128, 128)]
            ci = s * 128 + lane
            va, ia = _insert(x, ci, va, ia)
        return va, ia

    vals0 = tuple(jnp.full((_BR, 128), jnp.inf, jnp.float32) for _ in range(_T))
    idxs0 = tuple(jnp.full((_BR, 128), _W, jnp.int32) for _ in range(_T))
    va, ia = jax.lax.fori_loop(0, _S // _BS, ins_body, (vals0, idxs0))

    def ext_body(g, carry):
        va, ia, acc = carry
        for j in range(4):
            t = g * 4 + j
            s_t, va, ia = _pop(va, ia, r2)
            acc = jnp.where(kcol == t, s_t, acc)
        return va, ia, acc

    acc0 = jnp.full((_BR, _K), -1, jnp.int32)
    _, _, acc = jax.lax.fori_loop(0, _K // 4, ext_body, (va, ia, acc0))

    rowk = jax.lax.broadcasted_iota(jnp.int32, (_BR, _K), 0) + i * _BR
    src_ref[...] = acc
    dst_ref[...] = jnp.where(acc >= 0, rowk, -1)


def kernel(feature, pos):
    pos_pad = jnp.pad(pos, ((0, _W - _N), (0, 0)), constant_values=100.0)
    pos_t = pos_pad.T
    src, dst = pl.pallas_call(
        _radius_topk_kernel,
        grid=(_GRID,),
        in_specs=[
            pl.BlockSpec((_BR, 3), lambda i: (i, 0)),
            pl.BlockSpec((3, _W), lambda i: (0, 0)),
        ],
        out_specs=[
            pl.BlockSpec((_BR, _K), lambda i: (i, 0)),
            pl.BlockSpec((_BR, _K), lambda i: (i, 0)),
        ],
        out_shape=[
            jax.ShapeDtypeStruct((_W, _K), jnp.int32),
            jax.ShapeDtypeStruct((_W, _K), jnp.int32),
        ],
        scratch_shapes=[pltpu.VMEM((_BR, _W), jnp.float32)],
    )(pos_pad, pos_t)
    edge_src = src[:_N].reshape(-1)
    edge_dst = dst[:_N].reshape(-1)
    return feature, pos, edge_src, edge_dst


# speedup vs baseline: 1.5021x; 1.1709x over previous
import jax
import jax.numpy as jnp
from jax.experimental import pallas as pl
from jax.experimental.pallas import tpu as pltpu

_N = 10000
_K = 32
_R = 0.1 * 0.999
_R2 = _R * _R

_BR = 128
_W = 10240
_GRID = _W // _BR
_S = _W // 128
_BS = 40
_T = 9
_RG = 8


def _insert(v, ci, vals, idxs):
    new_vals, new_idxs = [], []
    c_prev = None
    for t in range(_T):
        c_t = v < vals[t]
        if t == 0:
            nv = jnp.where(c_t, v, vals[t])
            ni = jnp.where(c_t, ci, idxs[t])
        else:
            nv = jnp.where(c_t, jnp.where(c_prev, vals[t - 1], v), vals[t])
            ni = jnp.where(c_t, jnp.where(c_prev, idxs[t - 1], ci), idxs[t])
        new_vals.append(nv)
        new_idxs.append(ni)
        c_prev = c_t
    return tuple(new_vals), tuple(new_idxs)


def _pop(vals, idxs, r2):
    mval = jnp.min(vals[0], axis=1, keepdims=True)
    is_min = vals[0] == mval
    li = jnp.min(jnp.where(is_min, idxs[0], _W), axis=1, keepdims=True)
    pop = is_min & (idxs[0] == li)
    src_t = jnp.where(mval <= r2, li, -1)
    new_vals = tuple(jnp.where(pop, vals[u + 1], vals[u]) for u in range(_T - 1)) \
        + (jnp.where(pop, jnp.inf, vals[_T - 1]),)
    new_idxs = tuple(jnp.where(pop, idxs[u + 1], idxs[u]) for u in range(_T - 1)) \
        + (jnp.where(pop, _W, idxs[_T - 1]),)
    return src_t, new_vals, new_idxs


def _radius_topk_kernel(pos_r_ref, pos_t_ref, src_ref, dst_ref, work_ref):
    i = pl.program_id(0)
    pos_r = pos_r_ref[...]
    pos_t = pos_t_ref[...]
    sq_r = jnp.sum(pos_r * pos_r, axis=1, keepdims=True)
    sq_c = jnp.sum(pos_t * pos_t, axis=0, keepdims=True)
    m = jnp.dot(pos_r, pos_t, preferred_element_type=jnp.float32)
    d2 = sq_r + sq_c - 2.0 * m
    d2 = jnp.maximum(d2, 0.0)

    col = jax.lax.broadcasted_iota(jnp.int32, (_BR, _W), 1)
    row = jax.lax.broadcasted_iota(jnp.int32, (_BR, _W), 0) + i * _BR
    r2 = jnp.float32(_R2)
    valid = (d2 <= r2) & (col != row)
    work_ref[...] = jnp.where(valid, d2, jnp.inf)

    lane = jax.lax.broadcasted_iota(jnp.int32, (_BR, 128), 1)
    kcol = jax.lax.broadcasted_iota(jnp.int32, (_BR, _K), 1)

    def ins_body(g, carry):
        va, ia = carry
        for j in range(_BS):
            s = g * _BS + j
            x = work_ref[:, pl.ds(s * 128, 128)]
            ci = s * 128 + lane
            va, ia = _insert(x, ci, va, ia)
        return va, ia

    vals0 = tuple(jnp.full((_BR, 128), jnp.inf, jnp.float32) for _ in range(_T))
    idxs0 = tuple(jnp.full((_BR, 128), _W, jnp.int32) for _ in range(_T))
    va, ia = jax.lax.fori_loop(0, _S // _BS, ins_body, (vals0, idxs0))

    acc = jnp.full((_BR, _K), -1, jnp.int32)
    for t in range(_K):
        s_t, va, ia = _pop(va, ia, r2)
        acc = jnp.where(kcol == t, s_t, acc)

    rowk = jax.lax.broadcasted_iota(jnp.int32, (_BR, _K), 0) + i * _BR
    src_ref[...] = acc
    dst_ref[...] = jnp.where(acc >= 0, rowk, -1)


def kernel(feature, pos):
    pos_pad = jnp.pad(pos, ((0, _W - _N), (0, 0)), constant_values=100.0)
    pos_t = pos_pad.T
    src, dst = pl.pallas_call(
        _radius_topk_kernel,
        grid=(_GRID,),
        in_specs=[
            pl.BlockSpec((_BR, 3), lambda i: (i, 0)),
            pl.BlockSpec((3, _W), lambda i: (0, 0)),
        ],
        out_specs=[
            pl.BlockSpec((_BR, _K), lambda i: (i, 0)),
            pl.BlockSpec((_BR, _K), lambda i: (i, 0)),
        ],
        out_shape=[
            jax.ShapeDtypeStruct((_W, _K), jnp.int32),
            jax.ShapeDtypeStruct((_W, _K), jnp.int32),
        ],
        scratch_shapes=[pltpu.VMEM((_BR, _W), jnp.float32)],
    )(pos_pad, pos_t)
    edge_src = src[:_N].reshape(-1)
    edge_dst = dst[:_N].reshape(-1)
    return feature, pos, edge_src, edge_dst


# speedup vs baseline: 1.7846x; 1.1881x over previous
import jax
import jax.numpy as jnp
from jax.experimental import pallas as pl
from jax.experimental.pallas import tpu as pltpu

_N = 10000
_K = 32
_R = 0.1 * 0.999
_R2 = _R * _R

_BR = 128
_W = 10240
_GRID = _W // _BR
_S = _W // 128
_BS = 40
_T = 9
_RG = 8


def _insert(v, ci, vals, idxs):
    new_vals, new_idxs = [], []
    c_prev = None
    for t in range(_T):
        c_t = v < vals[t]
        if t == 0:
            nv = jnp.where(c_t, v, vals[t])
            ni = jnp.where(c_t, ci, idxs[t])
        else:
            nv = jnp.where(c_t, jnp.where(c_prev, vals[t - 1], v), vals[t])
            ni = jnp.where(c_t, jnp.where(c_prev, idxs[t - 1], ci), idxs[t])
        new_vals.append(nv)
        new_idxs.append(ni)
        c_prev = c_t
    return tuple(new_vals), tuple(new_idxs)


def _pop(vals, idxs, r2):
    mval = jnp.min(vals[0], axis=1, keepdims=True)
    is_min = vals[0] == mval
    li = jnp.min(jnp.where(is_min, idxs[0], _W), axis=1, keepdims=True)
    pop = is_min & (idxs[0] == li)
    src_t = jnp.where(mval <= r2, li, -1)
    new_vals = tuple(jnp.where(pop, vals[u + 1], vals[u]) for u in range(_T - 1)) \
        + (jnp.where(pop, jnp.inf, vals[_T - 1]),)
    new_idxs = tuple(jnp.where(pop, idxs[u + 1], idxs[u]) for u in range(_T - 1)) \
        + (jnp.where(pop, _W, idxs[_T - 1]),)
    return src_t, new_vals, new_idxs


def _radius_topk_kernel(pos_r_ref, pos_t_ref, src_ref, dst_ref, work_ref):
    i = pl.program_id(0)
    pos_r = pos_r_ref[...]
    pos_t = pos_t_ref[...]
    sq_r = jnp.sum(pos_r * pos_r, axis=1, keepdims=True)
    sq_c = jnp.sum(pos_t * pos_t, axis=0, keepdims=True)
    m = jnp.dot(pos_r, pos_t, preferred_element_type=jnp.float32)
    d2 = sq_r + sq_c - 2.0 * m
    d2 = jnp.maximum(d2, 0.0)

    col = jax.lax.broadcasted_iota(jnp.int32, (_BR, _W), 1)
    row = jax.lax.broadcasted_iota(jnp.int32, (_BR, _W), 0) + i * _BR
    r2 = jnp.float32(_R2)
    valid = (d2 <= r2) & (col != row)
    work_ref[...] = jnp.where(valid, d2, jnp.inf)

    lane = jax.lax.broadcasted_iota(jnp.int32, (_BR, 128), 1)

    va = tuple(jnp.full((_BR, 128), jnp.inf, jnp.float32) for _ in range(_T))
    ia = tuple(jnp.full((_BR, 128), _W, jnp.int32) for _ in range(_T))
    for s in range(_S):
        x = work_ref[:, s * 128:(s + 1) * 128]
        ci = s * 128 + lane
        va, ia = _insert(x, ci, va, ia)

    cols = []
    for t in range(_K):
        s_t, va, ia = _pop(va, ia, r2)
        cols.append(s_t)
    acc = jnp.concatenate(cols, axis=1)

    rowk = jax.lax.broadcasted_iota(jnp.int32, (_BR, _K), 0) + i * _BR
    src_ref[...] = acc
    dst_ref[...] = jnp.where(acc >= 0, rowk, -1)


def kernel(feature, pos):
    pos_pad = jnp.pad(pos, ((0, _W - _N), (0, 0)), constant_values=100.0)
    pos_t = pos_pad.T
    src, dst = pl.pallas_call(
        _radius_topk_kernel,
        grid=(_GRID,),
        in_specs=[
            pl.BlockSpec((_BR, 3), lambda i: (i, 0)),
            pl.BlockSpec((3, _W), lambda i: (0, 0)),
        ],
        out_specs=[
            pl.BlockSpec((_BR, _K), lambda i: (i, 0)),
            pl.BlockSpec((_BR, _K), lambda i: (i, 0)),
        ],
        out_shape=[
            jax.ShapeDtypeStruct((_W, _K), jnp.int32),
            jax.ShapeDtypeStruct((_W, _K), jnp.int32),
        ],
        scratch_shapes=[pltpu.VMEM((_BR, _W), jnp.float32)],
    )(pos_pad, pos_t)
    edge_src = src[:_N].reshape(-1)
    edge_dst = dst[:_N].reshape(-1)
    return feature, pos, edge_src, edge_dst


# speedup vs baseline: 1.7857x; 1.0006x over previous
import jax
import jax.numpy as jnp
from jax.experimental import pallas as pl
from jax.experimental.pallas import tpu as pltpu

_N = 10000
_K = 32
_R = 0.1 * 0.999
_R2 = _R * _R

_BR = 128
_W = 10240
_GRID = _W // _BR
_S = _W // 128
_T = 9


def _insert(v, ci, vals, idxs):
    new_vals, new_idxs = [], []
    c_prev = None
    for t in range(_T):
        c_t = v < vals[t]
        if t == 0:
            nv = jnp.where(c_t, v, vals[t])
            ni = jnp.where(c_t, ci, idxs[t])
        else:
            nv = jnp.where(c_t, jnp.where(c_prev, vals[t - 1], v), vals[t])
            ni = jnp.where(c_t, jnp.where(c_prev, idxs[t - 1], ci), idxs[t])
        new_vals.append(nv)
        new_idxs.append(ni)
        c_prev = c_t
    return tuple(new_vals), tuple(new_idxs)


def _pop(vals, idxs, r2):
    mval = jnp.min(vals[0], axis=1, keepdims=True)
    is_min = vals[0] == mval
    li = jnp.min(jnp.where(is_min, idxs[0], _W), axis=1, keepdims=True)
    pop = is_min & (idxs[0] == li)
    src_t = jnp.where(mval <= r2, li, -1)
    new_vals = tuple(jnp.where(pop, vals[u + 1], vals[u]) for u in range(_T - 1)) \
        + (jnp.where(pop, jnp.inf, vals[_T - 1]),)
    new_idxs = tuple(jnp.where(pop, idxs[u + 1], idxs[u]) for u in range(_T - 1)) \
        + (jnp.where(pop, _W, idxs[_T - 1]),)
    return src_t, new_vals, new_idxs


def _radius_topk_kernel(pos_r_ref, pos_t_ref, src_ref, dst_ref, work_ref):
    i = pl.program_id(0)
    pos_r = pos_r_ref[...]
    pos_t = pos_t_ref[...]
    sq_r = jnp.sum(pos_r * pos_r, axis=1, keepdims=True)
    sq_c = jnp.sum(pos_t * pos_t, axis=0, keepdims=True)
    m = jnp.dot(pos_r, pos_t, preferred_element_type=jnp.float32)
    d2 = sq_r + sq_c - 2.0 * m
    d2 = jnp.maximum(d2, 0.0)

    col = jax.lax.broadcasted_iota(jnp.int32, (_BR, _W), 1)
    row = jax.lax.broadcasted_iota(jnp.int32, (_BR, _W), 0) + i * _BR
    r2 = jnp.float32(_R2)
    valid = (d2 <= r2) & (col != row)
    work_ref[...] = jnp.where(valid, d2, jnp.inf)

    lane = jax.lax.broadcasted_iota(jnp.int32, (_BR, 128), 1)

    va = tuple(jnp.full((_BR, 128), jnp.inf, jnp.float32) for _ in range(_T))
    ia = tuple(jnp.full((_BR, 128), _W, jnp.int32) for _ in range(_T))
    for s in range(_S):
        x = work_ref[:, s * 128:(s + 1) * 128]
        ci = s * 128 + lane
        va, ia = _insert(x, ci, va, ia)

    cols = []
    for t in range(_K):
        s_t, va, ia = _pop(va, ia, r2)
        cols.append(s_t)
    acc = jnp.concatenate(cols, axis=1)

    rowk = jax.lax.broadcasted_iota(jnp.int32, (_BR, _K), 0) + i * _BR
    src_ref[...] = acc
    dst_ref[...] = jnp.where(acc >= 0, rowk, -1)


def kernel(feature, pos):
    pos_pad = jnp.pad(pos, ((0, _W - _N), (0, 0)), constant_values=100.0)
    pos_t = pos_pad.T
    src, dst = pl.pallas_call(
        _radius_topk_kernel,
        grid=(_GRID,),
        in_specs=[
            pl.BlockSpec((_BR, 3), lambda i: (i, 0)),
            pl.BlockSpec((3, _W), lambda i: (0, 0)),
        ],
        out_specs=[
            pl.BlockSpec((_BR, _K), lambda i: (i, 0)),
            pl.BlockSpec((_BR, _K), lambda i: (i, 0)),
        ],
        out_shape=[
            jax.ShapeDtypeStruct((_W, _K), jnp.int32),
            jax.ShapeDtypeStruct((_W, _K), jnp.int32),
        ],
        scratch_shapes=[pltpu.VMEM((_BR, _W), jnp.float32)],
    )(pos_pad, pos_t)
    edge_src = src[:_N].reshape(-1)
    edge_dst = dst[:_N].reshape(-1)
    return feature, pos, edge_src, edge_dst
